# Initial kernel scaffold; baseline (speedup 1.0000x reference)
import jax
import jax.numpy as jnp
from jax.experimental import pallas as pl


def kernel(x):
    raise NotImplementedError("write your pallas kernel here")


# speedup vs baseline: 2.1771x; 2.1771x over previous
import jax
import jax.numpy as jnp
from jax import lax
from jax.experimental import pallas as pl
from jax.experimental.pallas import tpu as pltpu

S_CHUNK = 256
F_BLK = 1024


def _cumsum_kernel(x_ref, o_ref, carry_ref):
    s = pl.program_id(2)

    @pl.when(s == 0)
    def _init():
        carry_ref[...] = jnp.zeros_like(carry_ref)

    ch = x_ref.shape[1]
    row = lax.broadcasted_iota(jnp.int32, (ch, ch), 0)
    col = lax.broadcasted_iota(jnp.int32, (ch, ch), 1)
    tril = (row >= col).astype(jnp.float32)
    local = jnp.dot(tril, x_ref[0], preferred_element_type=jnp.float32)
    out = local + carry_ref[...]
    o_ref[0] = out
    carry_ref[...] = out[-1:, :]


def kernel(x):
    B, S, F = x.shape
    grid = (B, F // F_BLK, S // S_CHUNK)
    return pl.pallas_call(
        _cumsum_kernel,
        grid=grid,
        in_specs=[pl.BlockSpec((1, S_CHUNK, F_BLK), lambda b, f, s: (b, s, f))],
        out_specs=pl.BlockSpec((1, S_CHUNK, F_BLK), lambda b, f, s: (b, s, f)),
        out_shape=jax.ShapeDtypeStruct(x.shape, x.dtype),
        scratch_shapes=[pltpu.VMEM((1, F_BLK), x.dtype)],
        compiler_params=pltpu.CompilerParams(
            dimension_semantics=("parallel", "parallel", "arbitrary"),
        ),
    )(x)


# speedup vs baseline: 2.1873x; 1.0047x over previous
import jax
import jax.numpy as jnp
from jax import lax
from jax.experimental import pallas as pl
from jax.experimental.pallas import tpu as pltpu

S_CHUNK = 256
F_BLK = 1024


def _cumsum_kernel(x_ref, o_ref, carry_ref):
    s = pl.program_id(2)

    @pl.when(s == 0)
    def _init():
        carry_ref[...] = jnp.zeros_like(carry_ref)

    ch = x_ref.shape[1]
    row = lax.broadcasted_iota(jnp.int32, (ch, ch), 0)
    col = lax.broadcasted_iota(jnp.int32, (ch, ch), 1)
    tril = (row >= col).astype(jnp.bfloat16)
    local = jnp.dot(tril, x_ref[0].astype(jnp.bfloat16),
                    preferred_element_type=jnp.float32)
    out = local + carry_ref[...]
    o_ref[0] = out
    carry_ref[...] = out[-1:, :]


def kernel(x):
    B, S, F = x.shape
    grid = (B, F // F_BLK, S // S_CHUNK)
    return pl.pallas_call(
        _cumsum_kernel,
        grid=grid,
        in_specs=[pl.BlockSpec((1, S_CHUNK, F_BLK), lambda b, f, s: (b, s, f))],
        out_specs=pl.BlockSpec((1, S_CHUNK, F_BLK), lambda b, f, s: (b, s, f)),
        out_shape=jax.ShapeDtypeStruct(x.shape, x.dtype),
        scratch_shapes=[pltpu.VMEM((1, F_BLK), x.dtype)],
        compiler_params=pltpu.CompilerParams(
            dimension_semantics=("parallel", "parallel", "arbitrary"),
        ),
    )(x)


# speedup vs baseline: 3.5923x; 1.6423x over previous
import jax
import jax.numpy as jnp
from jax import lax
from jax.experimental import pallas as pl
from jax.experimental.pallas import tpu as pltpu

S_CHUNK = 256
F_BLK = 4096


def _cumsum_kernel(x_ref, o_ref, carry_ref):
    s = pl.program_id(2)

    @pl.when(s == 0)
    def _init():
        carry_ref[...] = jnp.zeros_like(carry_ref)

    ch = x_ref.shape[1]
    row = lax.broadcasted_iota(jnp.int32, (ch, ch), 0)
    col = lax.broadcasted_iota(jnp.int32, (ch, ch), 1)
    tril = (row >= col).astype(jnp.bfloat16)
    local = jnp.dot(tril, x_ref[0].astype(jnp.bfloat16),
                    preferred_element_type=jnp.float32)
    out = local + carry_ref[...]
    o_ref[0] = out
    carry_ref[...] = out[-1:, :]


def kernel(x):
    B, S, F = x.shape
    grid = (B, F // F_BLK, S // S_CHUNK)
    return pl.pallas_call(
        _cumsum_kernel,
        grid=grid,
        in_specs=[pl.BlockSpec((1, S_CHUNK, F_BLK), lambda b, f, s: (b, s, f))],
        out_specs=pl.BlockSpec((1, S_CHUNK, F_BLK), lambda b, f, s: (b, s, f)),
        out_shape=jax.ShapeDtypeStruct(x.shape, x.dtype),
        scratch_shapes=[pltpu.VMEM((1, F_BLK), x.dtype)],
        compiler_params=pltpu.CompilerParams(
            dimension_semantics=("parallel", "parallel", "arbitrary"),
        ),
    )(x)
